# baseline (speedup 1.0000x reference)
import functools

import jax
import jax.numpy as jnp
from jax import lax
from jax.experimental import pallas as pl
from jax.experimental.pallas import tpu as pltpu
from jax.experimental.pallas import tpu_sc as plsc

_NUM_CORES = 2
_NUM_SUBCORES = 16
_NUM_WORKERS = _NUM_CORES * _NUM_SUBCORES

_NBUF = 8
_LANES = 16


def kernel(edge, table):
    batch, seq, _ = edge.shape
    rows, depth = table.shape
    n = batch * seq * seq
    idx_flat = edge.reshape(n).astype(jnp.int32)
    table_flat = table.reshape(rows * depth)

    per_worker = n // _NUM_WORKERS
    planes = per_worker // seq
    jgroups = seq // _LANES

    mesh = plsc.VectorSubcoreMesh(core_axis_name="c", subcore_axis_name="s")

    @functools.partial(
        pl.kernel,
        mesh=mesh,
        out_type=jax.ShapeDtypeStruct((batch, seq, depth, seq), jnp.float32),
        scratch_types=[
            pltpu.VMEM((per_worker,), jnp.int32),
            pltpu.VMEM((rows * depth,), jnp.float32),
            pltpu.VMEM((rows * (depth + 1),), jnp.float32),
            pltpu.VMEM((_NBUF, depth, seq), jnp.float32),
        ]
        + [pltpu.SemaphoreType.DMA] * _NBUF,
        compiler_params=pltpu.CompilerParams(
            use_tc_tiling_on_sc=False, needs_layout_passes=False
        ),
    )
    def lookup(table_hbm, idx_hbm, out_hbm, idx_v, tab_v, tabp_v, buf_v, *sem_w):
        wid = lax.axis_index("s") * _NUM_CORES + lax.axis_index("c")
        bat = wid // 2
        half = wid % 2
        base = wid * per_worker
        stride = depth + 1

        pltpu.sync_copy(table_hbm, tab_v)
        pltpu.sync_copy(idx_hbm.at[pl.ds(base, per_worker)], idx_v)

        for r in range(rows):
            for g in range(depth // _LANES):
                tabp_v[pl.ds(r * stride + g * _LANES, _LANES)] = tab_v[
                    pl.ds(r * depth + g * _LANES, _LANES)
                ]

        @pl.loop(0, per_worker, step=8 * _LANES)
        def _(k):
            for u in range(8):
                sl = pl.ds(k + u * _LANES, _LANES)
                idx_v[sl] = idx_v[sl] * stride

        @pl.loop(0, planes, step=_NBUF)
        def _(p0):
            for b in range(_NBUF):
                p = p0 + b

                @pl.when(p0 > 0)
                def _():
                    pltpu.make_async_copy(
                        buf_v.at[b],
                        out_hbm.at[0, 0],
                        sem_w[b],
                    ).wait()

                jidx = [
                    idx_v[pl.ds(p * seq + jg * _LANES, _LANES)]
                    for jg in range(jgroups)
                ]

                @plsc.parallel_loop(0, depth, step=1, unroll=8)
                def _(d):
                    for jg in range(jgroups):
                        vals = plsc.load_gather(tabp_v, [jidx[jg] + d])
                        buf_v[b, d, pl.ds(jg * _LANES, _LANES)] = vals

                pltpu.async_copy(
                    buf_v.at[b],
                    out_hbm.at[bat, half * planes + p],
                    sem_w[b],
                )

        for b in range(_NBUF):
            pltpu.make_async_copy(
                buf_v.at[b],
                out_hbm.at[0, 0],
                sem_w[b],
            ).wait()

    out = lookup(table_flat, idx_flat)
    return jnp.swapaxes(out, 2, 3)


# speedup vs baseline: 1.1772x; 1.1772x over previous
import functools

import jax
import jax.numpy as jnp
from jax import lax
from jax.experimental import pallas as pl
from jax.experimental.pallas import tpu as pltpu
from jax.experimental.pallas import tpu_sc as plsc

_NUM_CORES = 2
_NUM_SUBCORES = 16
_NUM_WORKERS = _NUM_CORES * _NUM_SUBCORES

_NBUF = 2
_LANES = 16


def kernel(edge, table):
    batch, seq, _ = edge.shape
    rows, depth = table.shape
    n = batch * seq * seq
    idx_flat = edge.reshape(n).astype(jnp.int32)
    table_flat = table.reshape(rows * depth)

    per_worker = n // _NUM_WORKERS
    planes = per_worker // seq
    jgroups = seq // _LANES

    mesh = plsc.VectorSubcoreMesh(core_axis_name="c", subcore_axis_name="s")

    @functools.partial(
        pl.kernel,
        mesh=mesh,
        out_type=jax.ShapeDtypeStruct((batch, seq, depth, seq), jnp.float32),
        scratch_types=[
            pltpu.VMEM((per_worker,), jnp.int32),
            pltpu.VMEM((rows * depth,), jnp.float32),
            pltpu.VMEM((rows * (depth + 1),), jnp.float32),
            pltpu.VMEM((_NBUF, depth, seq), jnp.float32),
        ]
        + [pltpu.SemaphoreType.DMA] * _NBUF,
        compiler_params=pltpu.CompilerParams(
            use_tc_tiling_on_sc=False, needs_layout_passes=False
        ),
    )
    def lookup(table_hbm, idx_hbm, out_hbm, idx_v, tab_v, tabp_v, buf_v, *sem_w):
        wid = lax.axis_index("s") * _NUM_CORES + lax.axis_index("c")
        bat = wid // 2
        half = wid % 2
        base = wid * per_worker
        stride = depth + 1

        pltpu.sync_copy(table_hbm, tab_v)
        pltpu.sync_copy(idx_hbm.at[pl.ds(base, per_worker)], idx_v)

        for r in range(rows):
            for g in range(depth // _LANES):
                tabp_v[pl.ds(r * stride + g * _LANES, _LANES)] = tab_v[
                    pl.ds(r * depth + g * _LANES, _LANES)
                ]

        @pl.loop(0, per_worker, step=8 * _LANES)
        def _(k):
            for u in range(8):
                sl = pl.ds(k + u * _LANES, _LANES)
                idx_v[sl] = idx_v[sl] * stride

        @pl.loop(0, planes, step=_NBUF)
        def _(p0):
            for b in range(_NBUF):
                p = p0 + b

                @pl.when(p0 > 0)
                def _():
                    pltpu.make_async_copy(
                        buf_v.at[b],
                        out_hbm.at[0, 0],
                        sem_w[b],
                    ).wait()

                jidx = [
                    idx_v[pl.ds(p * seq + jg * _LANES, _LANES)]
                    for jg in range(jgroups)
                ]

                @plsc.parallel_loop(0, depth, step=1, unroll=8)
                def _(d):
                    for jg in range(jgroups):
                        vals = plsc.load_gather(tabp_v, [jidx[jg] + d])
                        buf_v[b, d, pl.ds(jg * _LANES, _LANES)] = vals

                pltpu.async_copy(
                    buf_v.at[b],
                    out_hbm.at[bat, half * planes + p],
                    sem_w[b],
                )

        for b in range(_NBUF):
            pltpu.make_async_copy(
                buf_v.at[b],
                out_hbm.at[0, 0],
                sem_w[b],
            ).wait()

    out = lookup(table_flat, idx_flat)
    return jnp.swapaxes(out, 2, 3)
